# baseline (speedup 1.0000x reference)
import jax
import jax.numpy as jnp
from jax.experimental import pallas as pl

N = 4096
NC = 30
WIN = 100
D = 300
ATT_K = 25
MBLK = 64

NEG = -1e10


def _fused_body(emb_a, emb_b, cmask_ref, tok_a, tok_b, tmask_ref,
                b1_ref, b2_ref, out_ref):
    b2 = b2_ref[...]
    b1 = b1_ref[...]
    half = MBLK // 2

    ts_rows = []
    s2_list = []
    for m in range(MBLK):
        e = emb_a[m] if m < half else emb_b[m - half]
        t = tok_a[m] if m < half else tok_b[m - half]
        embc = jnp.concatenate([e * b2, e * b1], axis=0)
        s = jax.lax.dot_general(
            embc, t,
            dimension_numbers=(((1,), (1,)), ((), ())),
            preferred_element_type=jnp.float32)
        ts_rows.append(jnp.max(s[:NC], axis=0, keepdims=True))
        s2_list.append(s[NC:])
    ts = jnp.concatenate(ts_rows, axis=0)
    ts = jnp.where(tmask_ref[...] > 0, ts, NEG)
    tst = ts.T

    work = tst
    thr = jnp.max(work, axis=0, keepdims=True)
    m0 = thr
    for _ in range(ATT_K - 1):
        work = jnp.where(work >= thr, -jnp.inf, work)
        thr = jnp.max(work, axis=0, keepdims=True)

    ex = jnp.where(tst >= thr, jnp.exp(tst - m0), 0.0)
    probs_t = ex / jnp.sum(ex, axis=0, keepdims=True)
    probs = probs_t.T

    for m in range(MBLK):
        v = jax.lax.dot_general(
            probs[m:m + 1], s2_list[m],
            dimension_numbers=(((1,), (1,)), ((), ())),
            preferred_element_type=jnp.float32)
        out_ref[m, :] = jnp.where(cmask_ref[m, :] > 0, v[0], NEG)


@jax.jit
def _run(embeddings, cmask, token_embeddings, tmask, b1, b2):
    grid = (N // MBLK,)
    return pl.pallas_call(
        _fused_body,
        grid=grid,
        in_specs=[
            pl.BlockSpec((MBLK // 2, NC, D), lambda i: (2 * i, 0, 0)),
            pl.BlockSpec((MBLK // 2, NC, D), lambda i: (2 * i + 1, 0, 0)),
            pl.BlockSpec((MBLK, NC), lambda i: (i, 0)),
            pl.BlockSpec((MBLK // 2, WIN, D), lambda i: (2 * i, 0, 0)),
            pl.BlockSpec((MBLK // 2, WIN, D), lambda i: (2 * i + 1, 0, 0)),
            pl.BlockSpec((MBLK, WIN), lambda i: (i, 0)),
            pl.BlockSpec((1, D), lambda i: (0, 0)),
            pl.BlockSpec((1, D), lambda i: (0, 0)),
        ],
        out_specs=pl.BlockSpec((MBLK, NC), lambda i: (i, 0)),
        out_shape=jax.ShapeDtypeStruct((N, NC), jnp.float32),
    )(embeddings, embeddings, cmask, token_embeddings, token_embeddings,
      tmask, b1, b2)


def kernel(n, embeddings, masks, token_embeddings, token_masks, B_diag1, B_diag2):
    del n
    cmask = masks.astype(jnp.int32)
    tmask = token_masks.astype(jnp.int32)
    b1 = B_diag1.reshape(1, D)
    b2 = B_diag2.reshape(1, D)
    return _run(embeddings, cmask, token_embeddings, tmask, b1, b2)
